# baseline (speedup 1.0000x reference)
import functools

import jax
import jax.numpy as jnp
from jax import lax
from jax.experimental import pallas as pl
from jax.experimental.pallas import tpu as pltpu
from jax.experimental.pallas import tpu_sc as plsc

_NC = 2
_NS = 16
_NW = _NC * _NS
_CHUNK = 128


@functools.partial(jax.jit, static_argnames=("n_rows", "d", "seq_len"))
def _sc_embed(table, idx2d, pos, *, n_rows, d, seq_len):
    b_per_w = n_rows // _NW
    chunks_per_w = b_per_w // _CHUNK
    reps = b_per_w // seq_len
    h = d // 2

    mesh = plsc.VectorSubcoreMesh(
        core_axis_name="c", subcore_axis_name="s",
        num_cores=_NC, num_subcores=_NS)

    @functools.partial(
        pl.kernel,
        out_type=(jax.ShapeDtypeStruct((n_rows, h), jnp.float32),
                  jax.ShapeDtypeStruct((n_rows, h), jnp.float32)),
        mesh=mesh,
        scratch_types=[
            pltpu.VMEM((chunks_per_w, _CHUNK), jnp.int32),
            pltpu.VMEM((b_per_w, d), jnp.float32),
            pltpu.SemaphoreType.DMA,
        ],
        compiler_params=pltpu.CompilerParams(use_tc_tiling_on_sc=False),
    )
    def k(table_hbm, idx_hbm, pos_hbm, re_hbm, im_hbm, idx_v, rows_v, sem):
        wid = lax.axis_index("s") * _NC + lax.axis_index("c")
        base = wid * b_per_w
        pltpu.sync_copy(idx_hbm.at[pl.ds(wid * chunks_per_w, chunks_per_w), :],
                        idx_v)
        for r in range(reps):
            pltpu.sync_copy(pos_hbm, rows_v.at[pl.ds(r * seq_len, seq_len), :])
        copies = []
        for j in range(chunks_per_w):
            copies.append(pltpu.async_copy(
                table_hbm.at[idx_v.at[j]],
                rows_v.at[pl.ds(j * _CHUNK, _CHUNK), :],
                sem, add=True))
        for c in copies:
            c.wait()
        pltpu.sync_copy(rows_v.at[:, pl.ds(0, h)],
                        re_hbm.at[pl.ds(base, b_per_w), :])
        pltpu.sync_copy(rows_v.at[:, pl.ds(h, h)],
                        im_hbm.at[pl.ds(base, b_per_w), :])

    return k(table, idx2d, pos)


def kernel(x, token_table, pos_embedding):
    B, L = x.shape
    d = token_table.shape[1]
    n_rows = B * L
    idx2d = x.reshape(n_rows // _CHUNK, _CHUNK).astype(jnp.int32)
    pos = pos_embedding[0, :L, :]
    re, im = _sc_embed(token_table, idx2d, pos, n_rows=n_rows, d=d, seq_len=L)
    re = re.reshape(B, L, d // 2)
    im = im.reshape(B, L, d // 2)
    return jax.lax.complex(re, im)


# speedup vs baseline: 1.4477x; 1.4477x over previous
import functools

import jax
import jax.numpy as jnp
from jax import lax
from jax.experimental import pallas as pl
from jax.experimental.pallas import tpu as pltpu
from jax.experimental.pallas import tpu_sc as plsc

_NC = 2
_NS = 16
_NW = _NC * _NS
_LANES = 16
_TILE_R = 8


@functools.partial(jax.jit, static_argnames=("n_rows", "d", "seq_len"))
def _sc_embed(table, idx2d, pos, *, n_rows, d, seq_len):
    b_per_w = n_rows // _NW
    rows_per_w = b_per_w // 128
    n_groups = b_per_w // _LANES
    n_super = n_groups // 2
    h = d // 2
    nch = h // _LANES

    mesh = plsc.VectorSubcoreMesh(
        core_axis_name="c", subcore_axis_name="s",
        num_cores=_NC, num_subcores=_NS)

    scratch = [
        pltpu.VMEM((rows_per_w, 128), jnp.int32),
        pltpu.VMEM((seq_len, d), jnp.float32),
    ]
    scratch += [pltpu.VMEM((_TILE_R, d), jnp.float32)] * (2 * _LANES)
    scratch += [pltpu.VMEM((2 * _LANES, h), jnp.float32)] * 4
    scratch += [pltpu.SemaphoreType.DMA] * 4

    @functools.partial(
        pl.kernel,
        out_type=(jax.ShapeDtypeStruct((n_rows, h), jnp.float32),
                  jax.ShapeDtypeStruct((n_rows, h), jnp.float32)),
        mesh=mesh,
        scratch_types=scratch,
        compiler_params=pltpu.CompilerParams(
            use_tc_tiling_on_sc=True, needs_layout_passes=False),
    )
    def k(table_hbm, idx_hbm, pos_hbm, re_hbm, im_hbm,
          idx_v, pos_v, *bufs_sems):
        bank = (bufs_sems[:_LANES], bufs_sems[_LANES:2 * _LANES])
        st = bufs_sems[2 * _LANES:2 * _LANES + 4]
        stage = ((st[0], st[1]), (st[2], st[3]))
        sems = bufs_sems[2 * _LANES + 4:2 * _LANES + 6]
        sem_w = bufs_sems[2 * _LANES + 6:2 * _LANES + 8]
        wid = lax.axis_index("s") * _NC + lax.axis_index("c")
        base = wid * b_per_w
        pltpu.sync_copy(idx_hbm.at[pl.ds(wid * rows_per_w, rows_per_w), :],
                        idx_v)
        pltpu.sync_copy(pos_hbm, pos_v)

        lanes_iota = lax.iota(jnp.int32, _LANES)
        int_min = jnp.int32(-2**31)

        def group_vec(g):
            return idx_v[g >> 3, pl.ds((g & 7) * _LANES, _LANES)]

        def lane_scalar(vec, lane):
            return lax.reduce_max(
                jnp.where(lanes_iota == lane, vec, int_min), axes=(0,))

        def fire(g, b):
            vec = group_vec(g)
            for lane in range(_LANES):
                t = lane_scalar(vec, lane)
                tb = pl.multiple_of((t >> 3) * _TILE_R, _TILE_R)
                pltpu.async_copy(table_hbm.at[pl.ds(tb, _TILE_R), :],
                                 bank[b][lane], sems[b])

        def drain(b):
            for lane in range(_LANES):
                pltpu.make_async_copy(table_hbm.at[pl.ds(0, _TILE_R), :],
                                      bank[b][lane], sems[b]).wait()

        def process(g, b, p, half):
            vec = group_vec(g)
            for lane in range(_LANES):
                t = lane_scalar(vec, lane)
                r = jnp.bitwise_and(t, 7)
                lp = jnp.bitwise_and(g * _LANES + lane, seq_len - 1)
                buf = bank[b][lane]
                srow = half * _LANES + lane
                for c in range(nch):
                    s = pl.ds(c * _LANES, _LANES)
                    s2 = pl.ds(h + c * _LANES, _LANES)
                    stage[p][0][srow, s] = buf[r, s] + pos_v[lp, s]
                    stage[p][1][srow, s] = buf[r, s2] + pos_v[lp, s2]

        def stage_out(m, p):
            pltpu.async_copy(stage[p][0],
                             re_hbm.at[pl.ds(base + m * 2 * _LANES,
                                             2 * _LANES), :], sem_w[p])
            pltpu.async_copy(stage[p][1],
                             im_hbm.at[pl.ds(base + m * 2 * _LANES,
                                             2 * _LANES), :], sem_w[p])

        def stage_drain(p):
            for sref in (stage[p][0], stage[p][1]):
                pltpu.make_async_copy(
                    sref, re_hbm.at[pl.ds(0, 2 * _LANES), :],
                    sem_w[p]).wait()

        fire(0, 0)

        def body(m, _):
            g0 = m * 2
            p = jnp.bitwise_and(m, 1)

            fire(g0 + 1, 1)
            drain(0)

            @pl.when(p == 0)
            def _():
                @pl.when(m >= 2)
                def _():
                    stage_drain(0)
                process(g0, 0, 0, 0)

            @pl.when(p == 1)
            def _():
                @pl.when(m >= 2)
                def _():
                    stage_drain(1)
                process(g0, 0, 1, 0)

            @pl.when(m < n_super - 1)
            def _():
                fire(g0 + 2, 0)

            drain(1)

            @pl.when(p == 0)
            def _():
                process(g0 + 1, 1, 0, 1)
                stage_out(m, 0)

            @pl.when(p == 1)
            def _():
                process(g0 + 1, 1, 1, 1)
                stage_out(m, 1)

            return 0

        lax.fori_loop(0, n_super, body, 0)
        stage_drain(0)
        stage_drain(1)

    return k(table, idx2d, pos)


def kernel(x, token_table, pos_embedding):
    B, L = x.shape
    d = token_table.shape[1]
    n_rows = B * L
    idx2d = x.reshape(n_rows // 128, 128).astype(jnp.int32)
    pos = pos_embedding[0, :L, :]
    re, im = _sc_embed(token_table, idx2d, pos, n_rows=n_rows, d=d, seq_len=L)
    re = re.reshape(B, L, d // 2)
    im = im.reshape(B, L, d // 2)
    return jax.lax.complex(re, im)
